# baseline (speedup 1.0000x reference)
import functools

import numpy as np
import jax
import jax.numpy as jnp
from jax import lax
from jax.experimental import pallas as pl
from jax.experimental.pallas import tpu as pltpu
from jax.experimental.pallas import tpu_sc as plsc

_NODES = 10000
_EDGES = 320000
_HID = 128
_GRAPHS = 64
_NC = 2
_NS = 16
_NW = _NC * _NS
_EPW = _EDGES // _NW
_K = 40
_NCHUNK = _EPW // _K
_RPT = 624
_RTAIL = _NODES - _NS * _RPT

_MESH = plsc.VectorSubcoreMesh(core_axis_name="c", subcore_axis_name="s")


def _zero_rows(buf_v, sh_ref, sid, k):
  rbase = sid * _RPT
  nfull, rem = divmod(_RPT, k)
  for j in range(nfull):
    pltpu.sync_copy(buf_v.at[pl.ds(0, k)],
                    sh_ref.at[pl.ds(rbase + j * k, k)])
  if rem:
    pltpu.sync_copy(buf_v.at[pl.ds(0, rem)],
                    sh_ref.at[pl.ds(rbase + nfull * k, rem)])

  @pl.when(sid == _NS - 1)
  def _tail():
    pltpu.sync_copy(buf_v.at[pl.ds(0, _RTAIL)],
                    sh_ref.at[pl.ds(_NS * _RPT, _RTAIL)])


def _write_rows(sh_ref, out_ref, cid, sid):
  rbase = sid * _RPT
  pltpu.sync_copy(sh_ref.at[pl.ds(rbase, _RPT)],
                  out_ref.at[cid, pl.ds(rbase, _RPT)])

  @pl.when(sid == _NS - 1)
  def _tail():
    pltpu.sync_copy(sh_ref.at[pl.ds(_NS * _RPT, _RTAIL)],
                    out_ref.at[cid, pl.ds(_NS * _RPT, _RTAIL)])


def _make_edge_kernel():
  scratch = [
      pltpu.VMEM((_EPW,), jnp.int32),
      pltpu.VMEM((_EPW,), jnp.int32),
      [pltpu.VMEM((_K, _HID), jnp.float32)] * 3,
      [pltpu.VMEM((_K, _HID), jnp.float32)] * 3,
      pltpu.VMEM_SHARED((_NODES, _HID), jnp.float32),
      [pltpu.SemaphoreType.DMA] * 3,
      [pltpu.SemaphoreType.DMA] * 3,
      [pltpu.SemaphoreType.DMA] * 3,
  ]

  def body(a_hbm, b_hbm, src_hbm, dst_hbm, agg_out,
           src_w, dst_w, a_v, b_v, agg_sh, sem_a, sem_b, sem_s):
    cid = lax.axis_index("c")
    sid = lax.axis_index("s")
    wid = sid * _NC + cid
    zero16 = jnp.zeros((16,), jnp.float32)

    @pl.loop(0, _K)
    def _zrow(r):
      for c in range(_HID // 16):
        a_v[0][r, pl.ds(c * 16, 16)] = zero16

    _zero_rows(a_v[0], agg_sh, sid, _K)

    ebase = wid * _EPW
    pltpu.sync_copy(src_hbm.at[pl.ds(ebase, _EPW)], src_w)
    pltpu.sync_copy(dst_hbm.at[pl.ds(ebase, _EPW)], dst_w)
    plsc.subcore_barrier()

    def _didx(c):
      return dst_w.at[pl.ds(c * _K, _K)]

    def _sidx(c):
      return src_w.at[pl.ds(c * _K, _K)]

    def _prefetch(c, p):
      pltpu.async_copy(a_hbm.at[_didx(c)], a_v[p], sem_a[p])
      pltpu.async_copy(b_hbm.at[_sidx(c)], b_v[p], sem_b[p])

    def _wait_scatter(c, p):
      pltpu.make_async_copy(a_v[p], agg_sh.at[_didx(c)], sem_s[p]).wait()

    def _stage(c, p, prev_c=None):
      pltpu.make_async_copy(a_hbm.at[_didx(c)], a_v[p], sem_a[p]).wait()
      pltpu.make_async_copy(b_hbm.at[_sidx(c)], b_v[p], sem_b[p]).wait()

      @pl.loop(0, _K, unroll=2)
      def _crow(r):
        for cc in range(_HID // 16):
          s = pl.ds(cc * 16, 16)
          a_v[p][r, s] = jnp.maximum(a_v[p][r, s] + b_v[p][r, s], 0.0)

      pltpu.async_copy(a_v[p], agg_sh.at[_didx(c)], sem_s[p], add=True)
      if prev_c is not None:
        _wait_scatter(prev_c, (p + 2) % 3)

    _prefetch(0, 0)
    _prefetch(1, 1)
    _stage(0, 0)
    _prefetch(2, 2)
    _stage(1, 1, prev_c=0)
    _prefetch(3, 0)

    nloop = (_NCHUNK - 5) // 3

    @pl.loop(0, nloop)
    def _trip(i):
      c0 = 2 + 3 * i
      _stage(c0, 2, prev_c=c0 - 1)
      _prefetch(c0 + 2, 1)
      _stage(c0 + 1, 0, prev_c=c0)
      _prefetch(c0 + 3, 2)
      _stage(c0 + 2, 1, prev_c=c0 + 1)
      _prefetch(c0 + 4, 0)

    for c in range(2 + 3 * nloop, _NCHUNK):
      _stage(c, c % 3, prev_c=c - 1)
      if c + 2 < _NCHUNK:
        _prefetch(c + 2, (c + 2) % 3)
    _wait_scatter(_NCHUNK - 1, (_NCHUNK - 1) % 3)

    plsc.subcore_barrier()
    _write_rows(agg_sh, agg_out, cid, sid)

  return pl.kernel(
      body,
      out_type=jax.ShapeDtypeStruct((_NC, _NODES, _HID), jnp.float32),
      mesh=_MESH, scratch_types=scratch)


_KD = 128
_NCD = _EPW // _KD
_KDT = _EPW - _NCD * _KD


def _make_deg_kernel():
  scratch = [
      pltpu.VMEM((_EPW,), jnp.int32),
      pltpu.VMEM((_KD, _HID), jnp.float32),
      pltpu.VMEM_SHARED((_NODES, _HID), jnp.float32),
      pltpu.SemaphoreType.DMA,
  ]

  def body(dst_hbm, deg_out, dst_w, ones_v, deg_sh, sem_s):
    cid = lax.axis_index("c")
    sid = lax.axis_index("s")
    wid = sid * _NC + cid
    zero16 = jnp.zeros((16,), jnp.float32)

    @pl.loop(0, _KD)
    def _zrow(r):
      for c in range(_HID // 16):
        ones_v[r, pl.ds(c * 16, 16)] = zero16

    _zero_rows(ones_v, deg_sh, sid, _KD)

    pltpu.sync_copy(dst_hbm.at[pl.ds(wid * _EPW, _EPW)], dst_w)

    one16 = jnp.full((16,), 1.0, jnp.float32)

    @pl.loop(0, _KD)
    def _orow(r):
      for c in range(_HID // 16):
        ones_v[r, pl.ds(c * 16, 16)] = one16

    plsc.subcore_barrier()

    @pl.loop(0, _NCD)
    def _fire(j):
      pltpu.async_copy(ones_v, deg_sh.at[dst_w.at[pl.ds(j * _KD, _KD)]],
                       sem_s, add=True)

    pltpu.async_copy(ones_v.at[pl.ds(0, _KDT)],
                     deg_sh.at[dst_w.at[pl.ds(_NCD * _KD, _KDT)]],
                     sem_s, add=True)

    @pl.loop(0, _NCD)
    def _drain(j):
      pltpu.make_async_copy(
          ones_v, deg_sh.at[dst_w.at[pl.ds(j * _KD, _KD)]], sem_s).wait()

    pltpu.make_async_copy(
        ones_v.at[pl.ds(0, _KDT)],
        deg_sh.at[dst_w.at[pl.ds(_NCD * _KD, _KDT)]], sem_s).wait()

    plsc.subcore_barrier()
    _write_rows(deg_sh, deg_out, cid, sid)

  return pl.kernel(
      body,
      out_type=jax.ShapeDtypeStruct((_NC, _NODES, _HID), jnp.float32),
      mesh=_MESH, scratch_types=scratch)


_R = 1000


def _mm_ab(h, w1cat, b1r):
  def body(h_ref, w_ref, b_ref, a_ref, bo_ref):
    ab = jnp.dot(h_ref[...], w_ref[...], preferred_element_type=jnp.float32)
    a_ref[...] = ab[:, :_HID] + b_ref[...]
    bo_ref[...] = ab[:, _HID:]

  return pl.pallas_call(
      body,
      grid=(_NODES // _R,),
      in_specs=[
          pl.BlockSpec((_R, _HID), lambda i: (i, 0)),
          pl.BlockSpec((_HID, 2 * _HID), lambda i: (0, 0)),
          pl.BlockSpec((1, _HID), lambda i: (0, 0)),
      ],
      out_specs=[
          pl.BlockSpec((_R, _HID), lambda i: (i, 0)),
          pl.BlockSpec((_R, _HID), lambda i: (i, 0)),
      ],
      out_shape=[
          jax.ShapeDtypeStruct((_NODES, _HID), jnp.float32),
          jax.ShapeDtypeStruct((_NODES, _HID), jnp.float32),
      ],
  )(h, w1cat, b1r)


def _mm_out(agg2, deg2, w2, b2r, do_relu):
  def body(g_ref, d_ref, w_ref, b_ref, o_ref):
    a = g_ref[0] + g_ref[1]
    deg = d_ref[0, :, 0:1] + d_ref[1, :, 0:1]
    y = jnp.dot(a, w_ref[...], preferred_element_type=jnp.float32)
    y = y + deg * b_ref[...]
    if do_relu:
      y = jnp.maximum(y, 0.0)
    o_ref[...] = y

  return pl.pallas_call(
      body,
      grid=(_NODES // _R,),
      in_specs=[
          pl.BlockSpec((_NC, _R, _HID), lambda i: (0, i, 0)),
          pl.BlockSpec((_NC, _R, _HID), lambda i: (0, i, 0)),
          pl.BlockSpec((_HID, _HID), lambda i: (0, 0)),
          pl.BlockSpec((1, _HID), lambda i: (0, 0)),
      ],
      out_specs=pl.BlockSpec((_R, _HID), lambda i: (i, 0)),
      out_shape=jax.ShapeDtypeStruct((_NODES, _HID), jnp.float32),
  )(agg2, deg2, w2, b2r)


def _mm_mid(agg2, deg2, w2, b2r, w1cat, b1r):
  def body(g_ref, d_ref, w_ref, b_ref, w1_ref, b1_ref, a_ref, bo_ref):
    a = g_ref[0] + g_ref[1]
    deg = d_ref[0, :, 0:1] + d_ref[1, :, 0:1]
    y = jnp.dot(a, w_ref[...], preferred_element_type=jnp.float32)
    h = jnp.maximum(y + deg * b_ref[...], 0.0)
    ab = jnp.dot(h, w1_ref[...], preferred_element_type=jnp.float32)
    a_ref[...] = ab[:, :_HID] + b1_ref[...]
    bo_ref[...] = ab[:, _HID:]

  return pl.pallas_call(
      body,
      grid=(_NODES // _R,),
      in_specs=[
          pl.BlockSpec((_NC, _R, _HID), lambda i: (0, i, 0)),
          pl.BlockSpec((_NC, _R, _HID), lambda i: (0, i, 0)),
          pl.BlockSpec((_HID, _HID), lambda i: (0, 0)),
          pl.BlockSpec((1, _HID), lambda i: (0, 0)),
          pl.BlockSpec((_HID, 2 * _HID), lambda i: (0, 0)),
          pl.BlockSpec((1, _HID), lambda i: (0, 0)),
      ],
      out_specs=[
          pl.BlockSpec((_R, _HID), lambda i: (i, 0)),
          pl.BlockSpec((_R, _HID), lambda i: (i, 0)),
      ],
      out_shape=[
          jax.ShapeDtypeStruct((_NODES, _HID), jnp.float32),
          jax.ShapeDtypeStruct((_NODES, _HID), jnp.float32),
      ],
  )(agg2, deg2, w2, b2r, w1cat, b1r)


def _pool_head(h, batch_row, wf, bfr):
  ncls = wf.shape[1]

  def body(h_ref, b_ref, wf_ref, bf_ref, o_ref):
    gids = lax.broadcasted_iota(jnp.int32, (_GRAPHS, 1), 0)
    onehot = (b_ref[...] == gids).astype(jnp.float32)
    sums = jnp.dot(onehot, h_ref[...],
                   preferred_element_type=jnp.float32)
    cnt = jnp.sum(onehot, axis=1, keepdims=True)
    pooled = sums / jnp.maximum(cnt, 1.0)
    o_ref[...] = jnp.dot(pooled, wf_ref[...],
                         preferred_element_type=jnp.float32) + bf_ref[...]

  return pl.pallas_call(
      body,
      in_specs=[
          pl.BlockSpec((_NODES, _HID), lambda: (0, 0)),
          pl.BlockSpec((1, _NODES), lambda: (0, 0)),
          pl.BlockSpec((_HID, ncls), lambda: (0, 0)),
          pl.BlockSpec((1, ncls), lambda: (0, 0)),
      ],
      out_specs=pl.BlockSpec((_GRAPHS, ncls), lambda: (0, 0)),
      out_shape=jax.ShapeDtypeStruct((_GRAPHS, ncls), jnp.float32),
  )(h, batch_row, wf, bfr)


def kernel(x, edge_index, batch, params, Wf, bf):
  src = edge_index[0]
  dst = edge_index[1]
  edge = _make_edge_kernel()
  degk = _make_deg_kernel()

  deg2 = degk(dst)
  n = len(params)
  w1cats = []
  for (W1, b1, W2, b2) in params:
    d = W1.shape[0] // 2
    w1cats.append(jnp.concatenate([W1[:d], W1[d:]], axis=1))

  A, B = _mm_ab(x, w1cats[0], params[0][1].reshape(1, -1))
  for i in range(n):
    W1, b1, W2, b2 = params[i]
    agg2 = edge(A, B, src, dst)
    if i < n - 1:
      A, B = _mm_mid(agg2, deg2, W2, b2.reshape(1, -1),
                     w1cats[i + 1], params[i + 1][1].reshape(1, -1))
    else:
      h = _mm_out(agg2, deg2, W2, b2.reshape(1, -1), do_relu=False)
  return _pool_head(h, batch.reshape(1, -1), Wf, bf.reshape(1, -1))


# speedup vs baseline: 1.9598x; 1.9598x over previous
import functools

import numpy as np
import jax
import jax.numpy as jnp
from jax import lax
from jax.experimental import pallas as pl
from jax.experimental.pallas import tpu as pltpu
from jax.experimental.pallas import tpu_sc as plsc

_NODES = 10000
_EDGES = 320000
_HID = 128
_GRAPHS = 64
_NC = 2
_NS = 16
_NW = _NC * _NS
_EPW = _EDGES // _NW
_K = 40
_NCHUNK = _EPW // _K
_RPT = 624
_RTAIL = _NODES - _NS * _RPT

_MESH = plsc.VectorSubcoreMesh(core_axis_name="c", subcore_axis_name="s")


def _zero_rows(buf_v, sh_ref, sid, k):
  rbase = sid * _RPT
  nfull, rem = divmod(_RPT, k)
  for j in range(nfull):
    pltpu.sync_copy(buf_v.at[pl.ds(0, k)],
                    sh_ref.at[pl.ds(rbase + j * k, k)])
  if rem:
    pltpu.sync_copy(buf_v.at[pl.ds(0, rem)],
                    sh_ref.at[pl.ds(rbase + nfull * k, rem)])

  @pl.when(sid == _NS - 1)
  def _tail():
    pltpu.sync_copy(buf_v.at[pl.ds(0, _RTAIL)],
                    sh_ref.at[pl.ds(_NS * _RPT, _RTAIL)])


def _write_rows(sh_ref, out_ref, cid, sid):
  rbase = sid * _RPT
  pltpu.sync_copy(sh_ref.at[pl.ds(rbase, _RPT)],
                  out_ref.at[cid, pl.ds(rbase, _RPT)])

  @pl.when(sid == _NS - 1)
  def _tail():
    pltpu.sync_copy(sh_ref.at[pl.ds(_NS * _RPT, _RTAIL)],
                    out_ref.at[cid, pl.ds(_NS * _RPT, _RTAIL)])


def _make_edge_kernel():
  scratch = [
      pltpu.VMEM((_EPW,), jnp.int32),
      pltpu.VMEM((_EPW,), jnp.int32),
      [pltpu.VMEM((_K, _HID), jnp.float32)] * 3,
      [pltpu.VMEM((_K, _HID), jnp.float32)] * 3,
      pltpu.VMEM_SHARED((_NODES, _HID), jnp.float32),
      [pltpu.SemaphoreType.DMA] * 3,
      [pltpu.SemaphoreType.DMA] * 3,
      [pltpu.SemaphoreType.DMA] * 3,
  ]

  def body(a_hbm, b_hbm, src_hbm, dst_hbm, agg_out,
           src_w, dst_w, a_v, b_v, agg_sh, sem_a, sem_b, sem_s):
    cid = lax.axis_index("c")
    sid = lax.axis_index("s")
    wid = sid * _NC + cid
    zero16 = jnp.zeros((16,), jnp.float32)

    @pl.loop(0, _K)
    def _zrow(r):
      for c in range(_HID // 16):
        a_v[0][r, pl.ds(c * 16, 16)] = zero16

    _zero_rows(a_v[0], agg_sh, sid, _K)

    ebase = wid * _EPW
    pltpu.sync_copy(src_hbm.at[pl.ds(ebase, _EPW)], src_w)
    pltpu.sync_copy(dst_hbm.at[pl.ds(ebase, _EPW)], dst_w)
    plsc.subcore_barrier()

    def _didx(c):
      return dst_w.at[pl.ds(c * _K, _K)]

    def _sidx(c):
      return src_w.at[pl.ds(c * _K, _K)]

    def _prefetch(c, p):
      pltpu.async_copy(a_hbm.at[_didx(c)], a_v[p], sem_a[p])
      pltpu.async_copy(b_hbm.at[_sidx(c)], b_v[p], sem_b[p])

    def _wait_scatter(c, p):
      pltpu.make_async_copy(a_v[p], agg_sh.at[_didx(c)], sem_s[p]).wait()

    def _stage(c, p, prev_c=None):
      pltpu.make_async_copy(a_hbm.at[_didx(c)], a_v[p], sem_a[p]).wait()
      pltpu.make_async_copy(b_hbm.at[_sidx(c)], b_v[p], sem_b[p]).wait()

      @pl.loop(0, _K)
      def _crow(r):
        for cc in range(_HID // 16):
          s = pl.ds(cc * 16, 16)
          a_v[p][r, s] = jnp.maximum(a_v[p][r, s] + b_v[p][r, s], 0.0)

      pltpu.async_copy(a_v[p], agg_sh.at[_didx(c)], sem_s[p], add=True)
      if prev_c is not None:
        _wait_scatter(prev_c, (p + 2) % 3)

    _prefetch(0, 0)
    _prefetch(1, 1)
    _stage(0, 0)
    _prefetch(2, 2)
    _stage(1, 1, prev_c=0)
    _prefetch(3, 0)

    nloop = (_NCHUNK - 5) // 3

    @pl.loop(0, nloop)
    def _trip(i):
      c0 = 2 + 3 * i
      _stage(c0, 2, prev_c=c0 - 1)
      _prefetch(c0 + 2, 1)
      _stage(c0 + 1, 0, prev_c=c0)
      _prefetch(c0 + 3, 2)
      _stage(c0 + 2, 1, prev_c=c0 + 1)
      _prefetch(c0 + 4, 0)

    for c in range(2 + 3 * nloop, _NCHUNK):
      _stage(c, c % 3, prev_c=c - 1)
      if c + 2 < _NCHUNK:
        _prefetch(c + 2, (c + 2) % 3)
    _wait_scatter(_NCHUNK - 1, (_NCHUNK - 1) % 3)

    plsc.subcore_barrier()
    _write_rows(agg_sh, agg_out, cid, sid)

  return pl.kernel(
      body,
      out_type=jax.ShapeDtypeStruct((_NC, _NODES, _HID), jnp.float32),
      mesh=_MESH, scratch_types=scratch)


_KD = 128
_NCD = _EPW // _KD
_KDT = _EPW - _NCD * _KD


def _make_deg_kernel():
  scratch = [
      pltpu.VMEM((_EPW,), jnp.int32),
      pltpu.VMEM((_KD, _HID), jnp.float32),
      pltpu.VMEM_SHARED((_NODES, _HID), jnp.float32),
      pltpu.SemaphoreType.DMA,
  ]

  def body(dst_hbm, deg_out, dst_w, ones_v, deg_sh, sem_s):
    cid = lax.axis_index("c")
    sid = lax.axis_index("s")
    wid = sid * _NC + cid
    zero16 = jnp.zeros((16,), jnp.float32)

    @pl.loop(0, _KD)
    def _zrow(r):
      for c in range(_HID // 16):
        ones_v[r, pl.ds(c * 16, 16)] = zero16

    _zero_rows(ones_v, deg_sh, sid, _KD)

    pltpu.sync_copy(dst_hbm.at[pl.ds(wid * _EPW, _EPW)], dst_w)

    one16 = jnp.full((16,), 1.0, jnp.float32)

    @pl.loop(0, _KD)
    def _orow(r):
      for c in range(_HID // 16):
        ones_v[r, pl.ds(c * 16, 16)] = one16

    plsc.subcore_barrier()

    @pl.loop(0, _NCD)
    def _fire(j):
      pltpu.async_copy(ones_v, deg_sh.at[dst_w.at[pl.ds(j * _KD, _KD)]],
                       sem_s, add=True)

    pltpu.async_copy(ones_v.at[pl.ds(0, _KDT)],
                     deg_sh.at[dst_w.at[pl.ds(_NCD * _KD, _KDT)]],
                     sem_s, add=True)

    @pl.loop(0, _NCD)
    def _drain(j):
      pltpu.make_async_copy(
          ones_v, deg_sh.at[dst_w.at[pl.ds(j * _KD, _KD)]], sem_s).wait()

    pltpu.make_async_copy(
        ones_v.at[pl.ds(0, _KDT)],
        deg_sh.at[dst_w.at[pl.ds(_NCD * _KD, _KDT)]], sem_s).wait()

    plsc.subcore_barrier()
    _write_rows(deg_sh, deg_out, cid, sid)

  return pl.kernel(
      body,
      out_type=jax.ShapeDtypeStruct((_NC, _NODES, _HID), jnp.float32),
      mesh=_MESH, scratch_types=scratch)


_R = 1000


def _mm_ab(h, w1cat, b1r):
  def body(h_ref, w_ref, b_ref, a_ref, bo_ref):
    ab = jnp.dot(h_ref[...], w_ref[...], preferred_element_type=jnp.float32)
    a_ref[...] = ab[:, :_HID] + b_ref[...]
    bo_ref[...] = ab[:, _HID:]

  return pl.pallas_call(
      body,
      grid=(_NODES // _R,),
      in_specs=[
          pl.BlockSpec((_R, _HID), lambda i: (i, 0)),
          pl.BlockSpec((_HID, 2 * _HID), lambda i: (0, 0)),
          pl.BlockSpec((1, _HID), lambda i: (0, 0)),
      ],
      out_specs=[
          pl.BlockSpec((_R, _HID), lambda i: (i, 0)),
          pl.BlockSpec((_R, _HID), lambda i: (i, 0)),
      ],
      out_shape=[
          jax.ShapeDtypeStruct((_NODES, _HID), jnp.float32),
          jax.ShapeDtypeStruct((_NODES, _HID), jnp.float32),
      ],
  )(h, w1cat, b1r)


def _mm_out(agg2, deg2, w2, b2r, do_relu):
  def body(g_ref, d_ref, w_ref, b_ref, o_ref):
    a = g_ref[0] + g_ref[1]
    deg = d_ref[0, :, 0:1] + d_ref[1, :, 0:1]
    y = jnp.dot(a, w_ref[...], preferred_element_type=jnp.float32)
    y = y + deg * b_ref[...]
    if do_relu:
      y = jnp.maximum(y, 0.0)
    o_ref[...] = y

  return pl.pallas_call(
      body,
      grid=(_NODES // _R,),
      in_specs=[
          pl.BlockSpec((_NC, _R, _HID), lambda i: (0, i, 0)),
          pl.BlockSpec((_NC, _R, _HID), lambda i: (0, i, 0)),
          pl.BlockSpec((_HID, _HID), lambda i: (0, 0)),
          pl.BlockSpec((1, _HID), lambda i: (0, 0)),
      ],
      out_specs=pl.BlockSpec((_R, _HID), lambda i: (i, 0)),
      out_shape=jax.ShapeDtypeStruct((_NODES, _HID), jnp.float32),
  )(agg2, deg2, w2, b2r)


def _mm_mid(agg2, deg2, w2, b2r, w1cat, b1r):
  def body(g_ref, d_ref, w_ref, b_ref, w1_ref, b1_ref, a_ref, bo_ref):
    a = g_ref[0] + g_ref[1]
    deg = d_ref[0, :, 0:1] + d_ref[1, :, 0:1]
    y = jnp.dot(a, w_ref[...], preferred_element_type=jnp.float32)
    h = jnp.maximum(y + deg * b_ref[...], 0.0)
    ab = jnp.dot(h, w1_ref[...], preferred_element_type=jnp.float32)
    a_ref[...] = ab[:, :_HID] + b1_ref[...]
    bo_ref[...] = ab[:, _HID:]

  return pl.pallas_call(
      body,
      grid=(_NODES // _R,),
      in_specs=[
          pl.BlockSpec((_NC, _R, _HID), lambda i: (0, i, 0)),
          pl.BlockSpec((_NC, _R, _HID), lambda i: (0, i, 0)),
          pl.BlockSpec((_HID, _HID), lambda i: (0, 0)),
          pl.BlockSpec((1, _HID), lambda i: (0, 0)),
          pl.BlockSpec((_HID, 2 * _HID), lambda i: (0, 0)),
          pl.BlockSpec((1, _HID), lambda i: (0, 0)),
      ],
      out_specs=[
          pl.BlockSpec((_R, _HID), lambda i: (i, 0)),
          pl.BlockSpec((_R, _HID), lambda i: (i, 0)),
      ],
      out_shape=[
          jax.ShapeDtypeStruct((_NODES, _HID), jnp.float32),
          jax.ShapeDtypeStruct((_NODES, _HID), jnp.float32),
      ],
  )(agg2, deg2, w2, b2r, w1cat, b1r)


def _pool_head(h, batch_row, wf, bfr):
  ncls = wf.shape[1]

  def body(h_ref, b_ref, wf_ref, bf_ref, o_ref):
    gids = lax.broadcasted_iota(jnp.int32, (_GRAPHS, 1), 0)
    onehot = (b_ref[...] == gids).astype(jnp.float32)
    sums = jnp.dot(onehot, h_ref[...],
                   preferred_element_type=jnp.float32)
    cnt = jnp.sum(onehot, axis=1, keepdims=True)
    pooled = sums / jnp.maximum(cnt, 1.0)
    o_ref[...] = jnp.dot(pooled, wf_ref[...],
                         preferred_element_type=jnp.float32) + bf_ref[...]

  return pl.pallas_call(
      body,
      in_specs=[
          pl.BlockSpec((_NODES, _HID), lambda: (0, 0)),
          pl.BlockSpec((1, _NODES), lambda: (0, 0)),
          pl.BlockSpec((_HID, ncls), lambda: (0, 0)),
          pl.BlockSpec((1, ncls), lambda: (0, 0)),
      ],
      out_specs=pl.BlockSpec((_GRAPHS, ncls), lambda: (0, 0)),
      out_shape=jax.ShapeDtypeStruct((_GRAPHS, ncls), jnp.float32),
  )(h, batch_row, wf, bfr)


def kernel(x, edge_index, batch, params, Wf, bf):
  src = edge_index[0]
  dst = edge_index[1]
  edge = _make_edge_kernel()
  degk = _make_deg_kernel()

  deg2 = degk(dst)
  n = len(params)
  w1cats = []
  for (W1, b1, W2, b2) in params:
    d = W1.shape[0] // 2
    w1cats.append(jnp.concatenate([W1[:d], W1[d:]], axis=1))

  A, B = _mm_ab(x, w1cats[0], params[0][1].reshape(1, -1))
  for i in range(n):
    W1, b1, W2, b2 = params[i]
    agg2 = edge(A, B, src, dst)
    if i < n - 1:
      A, B = _mm_mid(agg2, deg2, W2, b2.reshape(1, -1),
                     w1cats[i + 1], params[i + 1][1].reshape(1, -1))
    else:
      h = _mm_out(agg2, deg2, W2, b2.reshape(1, -1), do_relu=False)
  return _pool_head(h, batch.reshape(1, -1), Wf, bf.reshape(1, -1))
